# Initial kernel scaffold; baseline (speedup 1.0000x reference)
import jax
import jax.numpy as jnp
from jax.experimental import pallas as pl


def kernel(inputs, embedding_table):
    raise NotImplementedError("write your pallas kernel here")


# speedup vs baseline: 1.4430x; 1.4430x over previous
import functools

import jax
import jax.numpy as jnp
from jax import lax
from jax.experimental import pallas as pl
from jax.experimental.pallas import tpu as pltpu
from jax.experimental.pallas import tpu_sc as plsc

VOCAB = 1000000
EMB = 32
CHUNK = 128
GROUP = 8


@functools.cache
def _build(B):
    info = plsc.get_sparse_core_info()
    NC, NS = info.num_cores, info.num_subcores
    NW = NC * NS
    assert B % (NW * CHUNK * GROUP) == 0
    b_per_w = B // NW
    n_chunks = b_per_w // CHUNK
    n_groups = n_chunks // GROUP
    rows_per_group = CHUNK * GROUP

    mesh = plsc.VectorSubcoreMesh(core_axis_name="c", subcore_axis_name="s")

    @functools.partial(
        pl.kernel,
        out_type=jax.ShapeDtypeStruct((B, EMB), jnp.float32),
        mesh=mesh,
        scratch_types=[
            pltpu.VMEM((n_chunks, CHUNK), jnp.int32),
            pltpu.VMEM((rows_per_group, EMB), jnp.float32),
            pltpu.SemaphoreType.DMA,
        ],
        compiler_params=pltpu.CompilerParams(use_tc_tiling_on_sc=False),
    )
    def k(idx_hbm, table_hbm, out_hbm, idx_v, rows_v, sem):
        wid = lax.axis_index("s") * NC + lax.axis_index("c")
        base = wid * b_per_w
        pltpu.sync_copy(idx_hbm.at[wid], idx_v)

        def group_body(g, carry):
            cps = []
            for b in range(GROUP):
                cps.append(pltpu.async_copy(
                    table_hbm.at[idx_v.at[g * GROUP + b]],
                    rows_v.at[pl.ds(b * CHUNK, CHUNK)],
                    sem))
            for cp in cps:
                cp.wait()
            pltpu.sync_copy(
                rows_v,
                out_hbm.at[pl.ds(base + g * rows_per_group, rows_per_group)])
            return carry

        lax.fori_loop(0, n_groups, group_body, 0)

    return k


def kernel(inputs, embedding_table):
    B_, L_ = inputs.shape
    B = B_ * L_
    info = plsc.get_sparse_core_info()
    NW = info.num_cores * info.num_subcores
    idx = inputs.reshape(NW, B // (NW * CHUNK), CHUNK).astype(jnp.int32)
    out = _build(B)(idx, embedding_table)
    return out.reshape(B_, L_, EMB), embedding_table


# speedup vs baseline: 1.4627x; 1.0137x over previous
import functools

import jax
import jax.numpy as jnp
from jax import lax
from jax.experimental import pallas as pl
from jax.experimental.pallas import tpu as pltpu
from jax.experimental.pallas import tpu_sc as plsc

VOCAB = 1000000
EMB = 32
CHUNK = 128
GROUP = 10


@functools.cache
def _build(B):
    info = plsc.get_sparse_core_info()
    NC, NS = info.num_cores, info.num_subcores
    NW = NC * NS
    assert B % (NW * CHUNK * GROUP) == 0
    b_per_w = B // NW
    n_chunks = b_per_w // CHUNK
    n_groups = n_chunks // GROUP
    RPG = CHUNK * GROUP
    assert n_groups % 2 == 0 and n_groups >= 4

    mesh = plsc.VectorSubcoreMesh(core_axis_name="c", subcore_axis_name="s")

    @functools.partial(
        pl.kernel,
        out_type=jax.ShapeDtypeStruct((B, EMB), jnp.float32),
        mesh=mesh,
        scratch_types=[
            pltpu.VMEM((n_chunks, CHUNK), jnp.int32),
            pltpu.VMEM((RPG, EMB), jnp.float32),
            pltpu.VMEM((RPG, EMB), jnp.float32),
            pltpu.SemaphoreType.DMA,
            pltpu.SemaphoreType.DMA,
            pltpu.SemaphoreType.DMA,
            pltpu.SemaphoreType.DMA,
        ],
        compiler_params=pltpu.CompilerParams(use_tc_tiling_on_sc=False),
    )
    def k(idx_hbm, table_hbm, out_hbm, idx_v, rows0, rows1,
          sg0, sg1, sw0, sw1):
        wid = lax.axis_index("s") * NC + lax.axis_index("c")
        base = wid * b_per_w
        pltpu.sync_copy(idx_hbm.at[wid], idx_v)

        def fire_gathers(g, buf, sem):
            for b in range(GROUP):
                pltpu.async_copy(
                    table_hbm.at[idx_v.at[g * GROUP + b]],
                    buf.at[pl.ds(b * CHUNK, CHUNK)],
                    sem)

        def wait_gathers(buf, sem):
            pltpu.make_async_copy(
                out_hbm.at[pl.ds(base, RPG)], buf, sem).wait()

        def fire_writeout(g, buf, sem):
            pltpu.async_copy(
                buf, out_hbm.at[pl.ds(base + g * RPG, RPG)], sem)

        def wait_writeout(buf, sem):
            pltpu.make_async_copy(
                buf, out_hbm.at[pl.ds(base, RPG)], sem).wait()

        fire_gathers(0, rows0, sg0)
        fire_gathers(1, rows1, sg1)
        wait_gathers(rows0, sg0)
        fire_writeout(0, rows0, sw0)

        def pair(t, carry):
            i1 = 2 * t + 1
            wait_writeout(rows0, sw0)
            fire_gathers(i1 + 1, rows0, sg0)
            wait_gathers(rows1, sg1)
            fire_writeout(i1, rows1, sw1)
            i2 = i1 + 1
            wait_writeout(rows1, sw1)
            fire_gathers(i2 + 1, rows1, sg1)
            wait_gathers(rows0, sg0)
            fire_writeout(i2, rows0, sw0)
            return carry

        lax.fori_loop(0, (n_groups - 2) // 2, pair, 0)

        wait_writeout(rows0, sw0)
        wait_gathers(rows1, sg1)
        fire_writeout(n_groups - 1, rows1, sw1)
        wait_writeout(rows1, sw1)

    return k


def kernel(inputs, embedding_table):
    B_, L_ = inputs.shape
    B = B_ * L_
    info = plsc.get_sparse_core_info()
    NW = info.num_cores * info.num_subcores
    idx = inputs.reshape(NW, B // (NW * CHUNK), CHUNK).astype(jnp.int32)
    out = _build(B)(idx, embedding_table)
    return out.reshape(B_, L_, EMB), embedding_table
